# baseline (speedup 1.0000x reference)
import jax
import jax.numpy as jnp
from jax import lax
from jax.experimental import pallas as pl
from jax.experimental.pallas import tpu as pltpu
from jax.experimental.pallas import tpu_sc as plsc

N = 10000
D = 128
E = 320000
NC, NS = 2, 16
NW = NC * NS
EPT = E // NW
CHUNK = 80
NCHUNK = EPT // CHUNK
RPT = 624


def _proj_body(x_ref, w_ref, ap_ref, y_ref, s_ref):
    y = jnp.dot(x_ref[...], w_ref[...], preferred_element_type=jnp.float32,
                precision=lax.Precision.HIGHEST)
    y_ref[...] = y
    s_ref[...] = jnp.dot(y, ap_ref[...], preferred_element_type=jnp.float32,
                         precision=lax.Precision.HIGHEST)


_PROJ_BM = 2000
_proj_call = pl.pallas_call(
    _proj_body,
    grid=(N // _PROJ_BM,),
    in_specs=[
        pl.BlockSpec((_PROJ_BM, D), lambda i: (i, 0)),
        pl.BlockSpec((D, D), lambda i: (0, 0)),
        pl.BlockSpec((D, 8), lambda i: (0, 0)),
    ],
    out_specs=[
        pl.BlockSpec((_PROJ_BM, D), lambda i: (i, 0)),
        pl.BlockSpec((_PROJ_BM, 8), lambda i: (i, 0)),
    ],
    out_shape=[
        jax.ShapeDtypeStruct((N, D), jnp.float32),
        jax.ShapeDtypeStruct((N, 8), jnp.float32),
    ],
)


def _edge_body(y_hbm, s1_hbm, s2_hbm, row_hbm, col_hbm,
               agg_out, den_out,
               row_i, col_i, s1_v, s2_v, ee_v, rows_v, zden_v,
               agg_sh, den_sh, sem_i, sem_g, sem_d, sem_a):
    cid = lax.axis_index("c")
    sid = lax.axis_index("s")
    wid = cid * NS + sid

    pltpu.sync_copy(s1_hbm, s1_v)
    pltpu.sync_copy(s2_hbm, s2_v)

    z16 = jnp.zeros((16,), jnp.float32)

    def _zero_rows(i, carry):
        for v in range(D // 16):
            rows_v[0, i, pl.ds(v * 16, 16)] = z16
        return carry

    lax.fori_loop(0, CHUNK, _zero_rows, 0)

    def _zero_zden(i, carry):
        zden_v[pl.ds(i * 16, 16)] = z16
        return carry

    lax.fori_loop(0, 63, _zero_zden, 0)

    base = sid * RPT
    for t in range(RPT // CHUNK):
        pltpu.sync_copy(rows_v.at[0], agg_sh.at[pl.ds(base + t * CHUNK, CHUNK)])
    rem = RPT - (RPT // CHUNK) * CHUNK
    pltpu.sync_copy(rows_v.at[0, pl.ds(0, rem)],
                    agg_sh.at[pl.ds(base + RPT - rem, rem)])

    @pl.when(sid < 2)
    def _():
        pltpu.sync_copy(rows_v.at[0, pl.ds(0, 8)],
                        agg_sh.at[pl.ds(NS * RPT + sid * 8, 8)])

    @pl.when(sid < 10)
    def _():
        pltpu.sync_copy(zden_v.at[pl.ds(0, 1000)],
                        den_sh.at[pl.ds(sid * 1000, 1000)])

    plsc.subcore_barrier()

    def _prepare(m, s, p, guard_retire, do_retire, guard_prefetch,
                 do_prefetch):
        rc, cc = row_i.at[s], col_i.at[s]
        rp, ep = rows_v.at[p], ee_v.at[p]

        pltpu.make_async_copy(row_hbm.at[wid * NCHUNK + m], rc, sem_i).wait()
        pltpu.make_async_copy(col_hbm.at[wid * NCHUNK + m], cc, sem_i).wait()

        def _retire_agg():
            pltpu.make_async_copy(rp, agg_sh.at[rc], sem_a).wait()

        def _retire_den():
            pltpu.make_async_copy(ep, den_sh.at[rc], sem_d).wait()

        if do_retire:
            if guard_retire:
                pl.when(m >= 2)(_retire_agg)
            else:
                _retire_agg()

        pltpu.async_copy(y_hbm.at[cc], rp, sem_g)

        if do_retire:
            if guard_retire:
                pl.when(m >= 2)(_retire_den)
            else:
                _retire_den()

        def _prefetch():
            sn = (s + 2) % 4
            pltpu.async_copy(row_hbm.at[wid * NCHUNK + m + 2],
                             row_i.at[sn], sem_i)
            pltpu.async_copy(col_hbm.at[wid * NCHUNK + m + 2],
                             col_i.at[sn], sem_i)

        if do_prefetch:
            pl.when(m < NCHUNK - 2)(_prefetch)

        for i in range(CHUNK // 16):
            rv = row_i[s, pl.ds(i * 16, 16)]
            cv = col_i[s, pl.ds(i * 16, 16)]
            e = plsc.load_gather(s1_v, [rv]) + plsc.load_gather(s2_v, [cv])
            e = jnp.where(e >= 0.0, e, 0.2 * e)
            ee_v[p, pl.ds(i * 16, 16)] = jnp.exp(e)
        pltpu.async_copy(ep, den_sh.at[rc], sem_d, add=True)

    def _finish(j, s, p):
        rc, cc = row_i.at[s], col_i.at[s]
        rp = rows_v.at[p]

        pltpu.make_async_copy(y_hbm.at[cc], rp, sem_g).wait()

        def _scale(g, c2):
            eev = ee_v[p, pl.ds(g * 16, 16)]
            for l in range(16):
                aa = eev[l]
                i = g * 16 + l
                for v in range(D // 16):
                    sl = pl.ds(v * 16, 16)
                    rows_v[p, i, sl] = rows_v[p, i, sl] * aa
            return c2

        lax.fori_loop(0, CHUNK // 16, _scale, 0)
        pltpu.async_copy(rp, agg_sh.at[rc], sem_a, add=True)

    pltpu.async_copy(row_hbm.at[wid * NCHUNK], row_i.at[0], sem_i)
    pltpu.async_copy(col_hbm.at[wid * NCHUNK], col_i.at[0], sem_i)
    pltpu.async_copy(row_hbm.at[wid * NCHUNK + 1], row_i.at[1], sem_i)
    pltpu.async_copy(col_hbm.at[wid * NCHUNK + 1], col_i.at[1], sem_i)
    _prepare(0, 0, 0, False, False, False, True)

    def _quad(k, carry):
        j = k * 4
        _prepare(j + 1, 1, 1, True, True, False, True)
        _finish(j, 0, 0)
        _prepare(j + 2, 2, 0, False, True, False, True)
        _finish(j + 1, 1, 1)
        _prepare(j + 3, 3, 1, False, True, False, True)
        _finish(j + 2, 2, 0)
        _prepare(j + 4, 0, 0, False, True, True, True)
        _finish(j + 3, 3, 1)
        return carry

    lax.fori_loop(0, (NCHUNK - 1) // 4, _quad, 0)
    _finish(NCHUNK - 1, 0, 0)

    for p in (0, 1):
        pltpu.make_async_copy(rows_v.at[p], agg_sh.at[row_i.at[0]],
                              sem_a).wait()
        pltpu.make_async_copy(ee_v.at[p], den_sh.at[row_i.at[0]],
                              sem_d).wait()

    plsc.subcore_barrier()

    pltpu.sync_copy(agg_sh.at[pl.ds(base, RPT)],
                    agg_out.at[cid, pl.ds(base, RPT)])

    @pl.when(sid < 2)
    def _():
        pltpu.sync_copy(agg_sh.at[pl.ds(NS * RPT + sid * 8, 8)],
                        agg_out.at[cid, pl.ds(NS * RPT + sid * 8, 8)])

    @pl.when(sid == 0)
    def _():
        pltpu.sync_copy(den_sh, den_out.at[cid])


_edge_call = pl.kernel(
    _edge_body,
    out_type=(
        jax.ShapeDtypeStruct((NC, N, D), jnp.float32),
        jax.ShapeDtypeStruct((NC, N), jnp.float32),
    ),
    mesh=plsc.VectorSubcoreMesh(core_axis_name="c", subcore_axis_name="s"),
    compiler_params=pltpu.CompilerParams(needs_layout_passes=False),
    scratch_types=[
        pltpu.VMEM((4, CHUNK), jnp.int32),
        pltpu.VMEM((4, CHUNK), jnp.int32),
        pltpu.VMEM((N,), jnp.float32),
        pltpu.VMEM((N,), jnp.float32),
        pltpu.VMEM((2, CHUNK), jnp.float32),
        pltpu.VMEM((2, CHUNK, D), jnp.float32),
        pltpu.VMEM((1008,), jnp.float32),
        pltpu.VMEM_SHARED((N, D), jnp.float32),
        pltpu.VMEM_SHARED((N,), jnp.float32),
        pltpu.SemaphoreType.DMA,
        pltpu.SemaphoreType.DMA,
        pltpu.SemaphoreType.DMA,
        pltpu.SemaphoreType.DMA,
    ],
)


def _final_body(agg_ref, den_ref, y_ref, o_ref):
    d = den_ref[0] + den_ref[1]
    d = jnp.where(d > 0.0, d, 1.0)
    x = (agg_ref[0] + agg_ref[1]) / d + y_ref[...]
    o_ref[...] = jnp.where(x > 0.0, x, jnp.exp(x) - 1.0)


_FIN_BM = 2000
_final_call = pl.pallas_call(
    _final_body,
    grid=(N // _FIN_BM,),
    in_specs=[
        pl.BlockSpec((NC, _FIN_BM, D), lambda i: (0, i, 0)),
        pl.BlockSpec((NC, _FIN_BM, 1), lambda i: (0, i, 0)),
        pl.BlockSpec((_FIN_BM, D), lambda i: (i, 0)),
    ],
    out_specs=pl.BlockSpec((_FIN_BM, D), lambda i: (i, 0)),
    out_shape=jax.ShapeDtypeStruct((N, D), jnp.float32),
)


@jax.jit
def kernel(input, triple, W, a):
    row3 = triple[:, 0].astype(jnp.int32).reshape(NW * NCHUNK, CHUNK)
    col3 = triple[:, 2].astype(jnp.int32).reshape(NW * NCHUNK, CHUNK)
    a_pad = jnp.zeros((D, 8), jnp.float32)
    a_pad = a_pad.at[:, 0].set(a[:D, 0]).at[:, 1].set(a[D:, 0])

    y, s12 = _proj_call(input.astype(jnp.float32), W.astype(jnp.float32), a_pad)
    s1 = s12[:, 0] + 0.0
    s2 = s12[:, 1] + 0.0

    agg2, den2 = _edge_call(y, s1, s2, row3, col3)
    return _final_call(agg2, den2.reshape(NC, N, 1), y)


# speedup vs baseline: 1.0424x; 1.0424x over previous
import jax
import jax.numpy as jnp
from jax import lax
from jax.experimental import pallas as pl
from jax.experimental.pallas import tpu as pltpu
from jax.experimental.pallas import tpu_sc as plsc

N = 10000
D = 128
E = 320000
NC, NS = 2, 16
NW = NC * NS
EPT = E // NW
CHUNK = 80
NCHUNK = EPT // CHUNK
RPT = 624


def _proj_body(x_ref, w_ref, ap_ref, y_ref, s_ref):
    y = jnp.dot(x_ref[...], w_ref[...], preferred_element_type=jnp.float32)
    y_ref[...] = y
    s_ref[...] = jnp.dot(y, ap_ref[...], preferred_element_type=jnp.float32)


_PROJ_BM = 2000
_proj_call = pl.pallas_call(
    _proj_body,
    grid=(N // _PROJ_BM,),
    in_specs=[
        pl.BlockSpec((_PROJ_BM, D), lambda i: (i, 0)),
        pl.BlockSpec((D, D), lambda i: (0, 0)),
        pl.BlockSpec((D, 8), lambda i: (0, 0)),
    ],
    out_specs=[
        pl.BlockSpec((_PROJ_BM, D), lambda i: (i, 0)),
        pl.BlockSpec((_PROJ_BM, 8), lambda i: (i, 0)),
    ],
    out_shape=[
        jax.ShapeDtypeStruct((N, D), jnp.float32),
        jax.ShapeDtypeStruct((N, 8), jnp.float32),
    ],
)


def _edge_body(y_hbm, s1_hbm, s2_hbm, row_hbm, col_hbm,
               agg_out, den_out,
               row_i, col_i, s1_v, s2_v, ee_v, rows_v, zden_v,
               agg_sh, den_sh, sem_i, sem_g, sem_d, sem_a):
    cid = lax.axis_index("c")
    sid = lax.axis_index("s")
    wid = cid * NS + sid

    pltpu.sync_copy(s1_hbm, s1_v)
    pltpu.sync_copy(s2_hbm, s2_v)

    z16 = jnp.zeros((16,), jnp.float32)

    def _zero_rows(i, carry):
        for v in range(D // 16):
            rows_v[0, i, pl.ds(v * 16, 16)] = z16
        return carry

    lax.fori_loop(0, CHUNK, _zero_rows, 0)

    def _zero_zden(i, carry):
        zden_v[pl.ds(i * 16, 16)] = z16
        return carry

    lax.fori_loop(0, 63, _zero_zden, 0)

    base = sid * RPT
    for t in range(RPT // CHUNK):
        pltpu.sync_copy(rows_v.at[0], agg_sh.at[pl.ds(base + t * CHUNK, CHUNK)])
    rem = RPT - (RPT // CHUNK) * CHUNK
    pltpu.sync_copy(rows_v.at[0, pl.ds(0, rem)],
                    agg_sh.at[pl.ds(base + RPT - rem, rem)])

    @pl.when(sid < 2)
    def _():
        pltpu.sync_copy(rows_v.at[0, pl.ds(0, 8)],
                        agg_sh.at[pl.ds(NS * RPT + sid * 8, 8)])

    @pl.when(sid < 10)
    def _():
        pltpu.sync_copy(zden_v.at[pl.ds(0, 1000)],
                        den_sh.at[pl.ds(sid * 1000, 1000)])

    plsc.subcore_barrier()

    def _prepare(m, s, p, guard_retire, do_retire, guard_prefetch,
                 do_prefetch):
        rc, cc = row_i.at[s], col_i.at[s]
        rp, ep = rows_v.at[p], ee_v.at[p]

        pltpu.make_async_copy(row_hbm.at[wid * NCHUNK + m], rc, sem_i).wait()
        pltpu.make_async_copy(col_hbm.at[wid * NCHUNK + m], cc, sem_i).wait()

        def _retire_agg():
            pltpu.make_async_copy(rp, agg_sh.at[rc], sem_a).wait()

        def _retire_den():
            pltpu.make_async_copy(ep, den_sh.at[rc], sem_d).wait()

        if do_retire:
            if guard_retire:
                pl.when(m >= 2)(_retire_agg)
            else:
                _retire_agg()

        pltpu.async_copy(y_hbm.at[cc], rp, sem_g)

        if do_retire:
            if guard_retire:
                pl.when(m >= 2)(_retire_den)
            else:
                _retire_den()

        def _prefetch():
            sn = (s + 2) % 4
            pltpu.async_copy(row_hbm.at[wid * NCHUNK + m + 2],
                             row_i.at[sn], sem_i)
            pltpu.async_copy(col_hbm.at[wid * NCHUNK + m + 2],
                             col_i.at[sn], sem_i)

        if do_prefetch:
            pl.when(m < NCHUNK - 2)(_prefetch)

        for i in range(CHUNK // 16):
            rv = row_i[s, pl.ds(i * 16, 16)]
            cv = col_i[s, pl.ds(i * 16, 16)]
            e = plsc.load_gather(s1_v, [rv]) + plsc.load_gather(s2_v, [cv])
            ee_v[p, pl.ds(i * 16, 16)] = jnp.exp(jnp.maximum(e, 0.2 * e))
        pltpu.async_copy(ep, den_sh.at[rc], sem_d, add=True)

    def _finish(j, s, p):
        rc, cc = row_i.at[s], col_i.at[s]
        rp = rows_v.at[p]

        pltpu.make_async_copy(y_hbm.at[cc], rp, sem_g).wait()

        def _scale(g, c2):
            eev = ee_v[p, pl.ds(g * 16, 16)]
            for l in range(16):
                aa = eev[l]
                i = g * 16 + l
                for v in range(D // 16):
                    sl = pl.ds(v * 16, 16)
                    rows_v[p, i, sl] = rows_v[p, i, sl] * aa
            return c2

        lax.fori_loop(0, CHUNK // 16, _scale, 0)
        pltpu.async_copy(rp, agg_sh.at[rc], sem_a, add=True)

    pltpu.async_copy(row_hbm.at[wid * NCHUNK], row_i.at[0], sem_i)
    pltpu.async_copy(col_hbm.at[wid * NCHUNK], col_i.at[0], sem_i)
    pltpu.async_copy(row_hbm.at[wid * NCHUNK + 1], row_i.at[1], sem_i)
    pltpu.async_copy(col_hbm.at[wid * NCHUNK + 1], col_i.at[1], sem_i)
    _prepare(0, 0, 0, False, False, False, True)

    def _quad(k, carry):
        j = k * 4
        _prepare(j + 1, 1, 1, True, True, False, True)
        _finish(j, 0, 0)
        _prepare(j + 2, 2, 0, False, True, False, True)
        _finish(j + 1, 1, 1)
        _prepare(j + 3, 3, 1, False, True, False, True)
        _finish(j + 2, 2, 0)
        _prepare(j + 4, 0, 0, False, True, True, True)
        _finish(j + 3, 3, 1)
        return carry

    lax.fori_loop(0, (NCHUNK - 1) // 4, _quad, 0)
    _finish(NCHUNK - 1, 0, 0)

    for p in (0, 1):
        pltpu.make_async_copy(rows_v.at[p], agg_sh.at[row_i.at[0]],
                              sem_a).wait()
        pltpu.make_async_copy(ee_v.at[p], den_sh.at[row_i.at[0]],
                              sem_d).wait()

    plsc.subcore_barrier()

    pltpu.sync_copy(agg_sh.at[pl.ds(base, RPT)],
                    agg_out.at[cid, pl.ds(base, RPT)])

    @pl.when(sid < 2)
    def _():
        pltpu.sync_copy(agg_sh.at[pl.ds(NS * RPT + sid * 8, 8)],
                        agg_out.at[cid, pl.ds(NS * RPT + sid * 8, 8)])

    @pl.when(sid == 0)
    def _():
        pltpu.sync_copy(den_sh, den_out.at[cid])


_edge_call = pl.kernel(
    _edge_body,
    out_type=(
        jax.ShapeDtypeStruct((NC, N, D), jnp.float32),
        jax.ShapeDtypeStruct((NC, N), jnp.float32),
    ),
    mesh=plsc.VectorSubcoreMesh(core_axis_name="c", subcore_axis_name="s"),
    compiler_params=pltpu.CompilerParams(needs_layout_passes=False),
    scratch_types=[
        pltpu.VMEM((4, CHUNK), jnp.int32),
        pltpu.VMEM((4, CHUNK), jnp.int32),
        pltpu.VMEM((N,), jnp.float32),
        pltpu.VMEM((N,), jnp.float32),
        pltpu.VMEM((2, CHUNK), jnp.float32),
        pltpu.VMEM((2, CHUNK, D), jnp.float32),
        pltpu.VMEM((1008,), jnp.float32),
        pltpu.VMEM_SHARED((N, D), jnp.float32),
        pltpu.VMEM_SHARED((N,), jnp.float32),
        pltpu.SemaphoreType.DMA,
        pltpu.SemaphoreType.DMA,
        pltpu.SemaphoreType.DMA,
        pltpu.SemaphoreType.DMA,
    ],
)


def _final_body(agg_ref, den_ref, y_ref, o_ref):
    d = den_ref[0] + den_ref[1]
    d = jnp.where(d > 0.0, d, 1.0)
    x = (agg_ref[0] + agg_ref[1]) / d + y_ref[...]
    o_ref[...] = jnp.where(x > 0.0, x, jnp.exp(x) - 1.0)


_FIN_BM = 2000
_final_call = pl.pallas_call(
    _final_body,
    grid=(N // _FIN_BM,),
    in_specs=[
        pl.BlockSpec((NC, _FIN_BM, D), lambda i: (0, i, 0)),
        pl.BlockSpec((NC, _FIN_BM, 1), lambda i: (0, i, 0)),
        pl.BlockSpec((_FIN_BM, D), lambda i: (i, 0)),
    ],
    out_specs=pl.BlockSpec((_FIN_BM, D), lambda i: (i, 0)),
    out_shape=jax.ShapeDtypeStruct((N, D), jnp.float32),
)


@jax.jit
def kernel(input, triple, W, a):
    row3 = triple[:, 0].astype(jnp.int32).reshape(NW * NCHUNK, CHUNK)
    col3 = triple[:, 2].astype(jnp.int32).reshape(NW * NCHUNK, CHUNK)
    a_pad = jnp.zeros((D, 8), jnp.float32)
    a_pad = a_pad.at[:, 0].set(a[:D, 0]).at[:, 1].set(a[D:, 0])

    y, s12 = _proj_call(input.astype(jnp.float32), W.astype(jnp.float32), a_pad)
    s1 = s12[:, 0] + 0.0
    s2 = s12[:, 1] + 0.0

    agg2, den2 = _edge_call(y, s1, s2, row3, col3)
    return _final_call(agg2, den2.reshape(NC, N, 1), y)
